# baseline (speedup 1.0000x reference)
import functools

import jax
import jax.numpy as jnp
from jax import lax
from jax.experimental import pallas as pl
from jax.experimental.pallas import tpu as pltpu
from jax.experimental.pallas import tpu_sc as plsc

N = 10000
E = 320000
D = 128

NC = 2
NS = 16
NW = NC * NS
FPT = D // NW
PPT = FPT // 2
EPW = E // NW
EC = 40000
L = 16
UNROLL = 8


@functools.cache
def _mesh():
    return plsc.VectorSubcoreMesh(core_axis_name="c", subcore_axis_name="s",
                                  num_cores=NC, num_subcores=NS)


def _wid():
    return lax.axis_index("s") * NC + lax.axis_index("c")


def _zero_fill(ref, nwords):
    z = jnp.zeros((L,), jnp.float32)

    def body(i, _):
        ref[pl.ds(i * L, L)] = z
        return 0

    lax.fori_loop(0, nwords // L, body, 0)


@functools.cache
def _hist_sc_kernel():
    return pl.kernel(
        _hist_sc_body,
        out_type=jax.ShapeDtypeStruct((NW, N), jnp.float32),
        mesh=_mesh(),
        scratch_types=[
            pltpu.VMEM((EPW,), jnp.int32),
            pltpu.VMEM((N,), jnp.float32),
        ],
        compiler_params=pltpu.CompilerParams(needs_layout_passes=False),
    )


def _hist_sc_body(pk_hbm, out_hbm, pk_v, hist_v):
    w = _wid()
    pltpu.sync_copy(pk_hbm.at[pl.ds(w * EPW, EPW)], pk_v)
    _zero_fill(hist_v, N)
    ones = jnp.ones((L,), jnp.float32)

    @plsc.parallel_loop(0, EPW // L, unroll=UNROLL)
    def body(j):
        cols = pk_v[pl.ds(j * L, L)] & 0xFFFF
        plsc.addupdate_scatter(hist_v, [cols], ones)

    pltpu.sync_copy(hist_v, out_hbm.at[w])


@functools.cache
def _agg_sc_kernel():
    return pl.kernel(
        _agg_sc_body,
        out_type=jax.ShapeDtypeStruct((NW, N * FPT), jnp.float32),
        mesh=_mesh(),
        scratch_types=[
            pltpu.VMEM((N * PPT,), jnp.int32),
            pltpu.VMEM((N * FPT,), jnp.float32),
            pltpu.VMEM((EC,), jnp.int32),
        ],
        compiler_params=pltpu.CompilerParams(needs_layout_passes=False),
    )


def _agg_sc_body(pk_hbm, lin_hbm, out_hbm, lin_v, acc_v, pk_v):
    w = _wid()
    pltpu.sync_copy(lin_hbm.at[w], lin_v)
    _zero_fill(acc_v, N * FPT)
    lin_p = [lin_v.at[pl.ds(p * N, N)] for p in range(PPT)]
    acc_f = [acc_v.at[pl.ds(f * N, N)] for f in range(FPT)]
    hi_mask = jnp.full((L,), -65536, jnp.int32)

    def chunk(c, _):
        pltpu.sync_copy(pk_hbm.at[pl.ds(c * EC, EC)], pk_v)

        @plsc.parallel_loop(0, EC // L, unroll=UNROLL)
        def grp(j):
            pk = pk_v[pl.ds(j * L, L)]
            rows = pk >> 16
            cols = pk & 0xFFFF
            for p in range(PPT):
                pair = plsc.load_gather(lin_p[p], [rows])
                lo = plsc.bitcast(pair << 16, jnp.float32)
                hi = plsc.bitcast(pair & hi_mask, jnp.float32)
                plsc.addupdate_scatter(acc_f[2 * p], [cols], lo)
                plsc.addupdate_scatter(acc_f[2 * p + 1], [cols], hi)

        return 0

    lax.fori_loop(0, E // EC, chunk, 0)
    pltpu.sync_copy(acc_v, out_hbm.at[w])


def _pack_body(row_ref, col_ref, pk_ref):
    pk_ref[...] = (row_ref[...] << 16) | col_ref[...]


def _pack(row, col):
    return pl.pallas_call(
        _pack_body,
        out_shape=jax.ShapeDtypeStruct((E,), jnp.int32),
    )(row, col)


def _deg_body(parts_ref, dis_ref, inv_ref):
    deg = jnp.sum(parts_ref[...], axis=0) + 1.0
    inv = 1.0 / deg
    inv_ref[...] = inv
    dis_ref[...] = jnp.sqrt(inv)


def _deg_finish(parts):
    return pl.pallas_call(
        _deg_body,
        out_shape=(
            jax.ShapeDtypeStruct((N,), jnp.float32),
            jax.ShapeDtypeStruct((N,), jnp.float32),
        ),
    )(parts)


def _pack_bf16_pairs(even, odd):
    a = lax.bitcast_convert_type(even.astype(jnp.bfloat16),
                                 jnp.uint16).astype(jnp.uint32)
    b = lax.bitcast_convert_type(odd.astype(jnp.bfloat16),
                                 jnp.uint16).astype(jnp.uint32)
    return lax.bitcast_convert_type((b << 16) | a, jnp.int32)


def _mm_body(x_ref, w_ref, we_ref, wo_ref, dis_ref, linT_ref, linsP_ref):
    dis = dis_ref[...].reshape(1, N)
    linT_ref[...] = lax.dot_general(
        w_ref[...], x_ref[...], (((0,), (1,)), ((), ())),
        preferred_element_type=jnp.float32)
    even = lax.dot_general(we_ref[...], x_ref[...], (((0,), (1,)), ((), ())),
                           preferred_element_type=jnp.float32) * dis
    odd = lax.dot_general(wo_ref[...], x_ref[...], (((0,), (1,)), ((), ())),
                          preferred_element_type=jnp.float32) * dis
    linsP_ref[...] = _pack_bf16_pairs(even, odd)


def _matmul(x, w, we, wo, dis):
    return pl.pallas_call(
        _mm_body,
        out_shape=(
            jax.ShapeDtypeStruct((D, N), jnp.float32),
            jax.ShapeDtypeStruct((D // 2, N), jnp.int32),
        ),
    )(x, w, we, wo, dis)


def _bn_relu_T(t, gamma, beta):
    m = jnp.mean(t, axis=1, keepdims=True)
    v = jnp.mean(t * t, axis=1, keepdims=True) - m * m
    h = (t - m) * lax.rsqrt(v + 1e-5) * gamma.reshape(D, 1) + beta.reshape(D, 1)
    return jnp.maximum(h, 0.0)


def _mid_body(accT_ref, linT_ref, dis_ref, inv_ref, b_ref, g_ref, be_ref,
              w2_ref, w2e_ref, w2o_ref, lin2T_ref, lin2sP_ref):
    dis = dis_ref[...].reshape(1, N)
    inv = inv_ref[...].reshape(1, N)
    t = dis * accT_ref[...] + inv * linT_ref[...] + b_ref[...].reshape(D, 1)
    h = _bn_relu_T(t, g_ref[...], be_ref[...])
    lin2T_ref[...] = lax.dot_general(w2_ref[...], h, (((0,), (0,)), ((), ())),
                                     preferred_element_type=jnp.float32)
    even = lax.dot_general(w2e_ref[...], h, (((0,), (0,)), ((), ())),
                           preferred_element_type=jnp.float32) * dis
    odd = lax.dot_general(w2o_ref[...], h, (((0,), (0,)), ((), ())),
                          preferred_element_type=jnp.float32) * dis
    lin2sP_ref[...] = _pack_bf16_pairs(even, odd)


def _layer_mid(accT, linT, dis, inv_deg, b, gamma, beta, w2, w2e, w2o):
    return pl.pallas_call(
        _mid_body,
        out_shape=(
            jax.ShapeDtypeStruct((D, N), jnp.float32),
            jax.ShapeDtypeStruct((D // 2, N), jnp.int32),
        ),
    )(accT, linT, dis, inv_deg, b, gamma, beta, w2, w2e, w2o)


def _out_body(accT_ref, linT_ref, dis_ref, inv_ref, b_ref, g_ref, be_ref, o_ref):
    dis = dis_ref[...].reshape(1, N)
    inv = inv_ref[...].reshape(1, N)
    t = dis * accT_ref[...] + inv * linT_ref[...] + b_ref[...].reshape(D, 1)
    h = _bn_relu_T(t, g_ref[...], be_ref[...])
    o_ref[...] = h.T


def _layer_out(accT, linT, dis, inv_deg, b, gamma, beta):
    return pl.pallas_call(
        _out_body,
        out_shape=jax.ShapeDtypeStruct((N, D), jnp.float32),
    )(accT, linT, dis, inv_deg, b, gamma, beta)


def kernel(x, edge_index, W1, b1, gamma1, beta1, W2, b2, gamma2, beta2):
    row = edge_index[0].astype(jnp.int32)
    col = edge_index[1].astype(jnp.int32)

    pk = _pack(row, col)
    parts = _hist_sc_kernel()(pk)
    dis, inv_deg = _deg_finish(parts)

    lin1T, lin1sP = _matmul(x, W1, W1[:, 0::2], W1[:, 1::2], dis)
    acc1T = _agg_sc_kernel()(pk, lin1sP.reshape(NW, N * PPT)).reshape(D, N)
    lin2T, lin2sP = _layer_mid(acc1T, lin1T, dis, inv_deg, b1, gamma1, beta1,
                               W2, W2[:, 0::2], W2[:, 1::2])
    acc2T = _agg_sc_kernel()(pk, lin2sP.reshape(NW, N * PPT)).reshape(D, N)
    return _layer_out(acc2T, lin2T, dis, inv_deg, b2, gamma2, beta2)


# speedup vs baseline: 1.1052x; 1.1052x over previous
import functools

import jax
import jax.numpy as jnp
from jax import lax
from jax.experimental import pallas as pl
from jax.experimental.pallas import tpu as pltpu
from jax.experimental.pallas import tpu_sc as plsc

N = 10000
E = 320000
D = 128

NC = 2
NS = 16
NW = NC * NS
FPT = D // NW
PPT = FPT // 2
EPW = E // NW
EC = 20000
L = 16
UNROLL = 4


@functools.cache
def _mesh():
    return plsc.VectorSubcoreMesh(core_axis_name="c", subcore_axis_name="s",
                                  num_cores=NC, num_subcores=NS)


def _wid():
    return lax.axis_index("s") * NC + lax.axis_index("c")


def _zero_fill(ref, nwords):
    z = jnp.zeros((L,), jnp.float32)

    def body(i, _):
        ref[pl.ds(i * L, L)] = z
        return 0

    lax.fori_loop(0, nwords // L, body, 0)


@functools.cache
def _hist_sc_kernel():
    return pl.kernel(
        _hist_sc_body,
        out_type=jax.ShapeDtypeStruct((NW, N), jnp.float32),
        mesh=_mesh(),
        scratch_types=[
            pltpu.VMEM((EPW,), jnp.int32),
            pltpu.VMEM((N,), jnp.float32),
        ],
        compiler_params=pltpu.CompilerParams(needs_layout_passes=False),
    )


def _hist_sc_body(pk_hbm, out_hbm, pk_v, hist_v):
    w = _wid()
    pltpu.sync_copy(pk_hbm.at[pl.ds(w * EPW, EPW)], pk_v)
    _zero_fill(hist_v, N)
    ones = jnp.ones((L,), jnp.float32)

    @plsc.parallel_loop(0, EPW // L, unroll=UNROLL)
    def body(j):
        cols = pk_v[pl.ds(j * L, L)] & 0xFFFF
        plsc.addupdate_scatter(hist_v, [cols], ones)

    pltpu.sync_copy(hist_v, out_hbm.at[w])


@functools.cache
def _agg_sc_kernel():
    return pl.kernel(
        _agg_sc_body,
        out_type=jax.ShapeDtypeStruct((NW, N * FPT), jnp.float32),
        mesh=_mesh(),
        scratch_types=[
            pltpu.VMEM((N * PPT,), jnp.int32),
            pltpu.VMEM((N * FPT,), jnp.float32),
            pltpu.VMEM((EC,), jnp.int32),
            pltpu.VMEM((EC,), jnp.int32),
            pltpu.SemaphoreType.DMA,
            pltpu.SemaphoreType.DMA,
        ],
        compiler_params=pltpu.CompilerParams(needs_layout_passes=False),
    )


def _agg_sc_body(pk_hbm, lin_hbm, out_hbm, lin_v, acc_v, pk_v0, pk_v1,
                 sem0, sem1):
    w = _wid()
    pltpu.sync_copy(lin_hbm.at[w], lin_v)
    _zero_fill(acc_v, N * FPT)
    lin_p = [lin_v.at[pl.ds(p * N, N)] for p in range(PPT)]
    acc_f = [acc_v.at[pl.ds(f * N, N)] for f in range(FPT)]
    hi_mask = jnp.full((L,), -65536, jnp.int32)
    bufs = [pk_v0, pk_v1]
    sems = [sem0, sem1]
    nchunk = E // EC

    cps = [pltpu.async_copy(pk_hbm.at[pl.ds(0, EC)], bufs[0], sems[0]), None]
    for c in range(nchunk):
        b = c & 1
        cps[b].wait()
        if c + 1 < nchunk:
            cps[1 - b] = pltpu.async_copy(
                pk_hbm.at[pl.ds((c + 1) * EC, EC)], bufs[1 - b], sems[1 - b])
        pk_v = bufs[b]

        @plsc.parallel_loop(0, EC // L, unroll=UNROLL)
        def grp(j):
            pk = pk_v[pl.ds(j * L, L)]
            rows = pk >> 16
            cols = pk & 0xFFFF
            for p in range(PPT):
                pair = plsc.load_gather(lin_p[p], [rows])
                lo = plsc.bitcast(pair << 16, jnp.float32)
                hi = plsc.bitcast(pair & hi_mask, jnp.float32)
                plsc.addupdate_scatter(acc_f[2 * p], [cols], lo)
                plsc.addupdate_scatter(acc_f[2 * p + 1], [cols], hi)

    pltpu.sync_copy(acc_v, out_hbm.at[w])


def _pack_body(row_ref, col_ref, pk_ref):
    pk_ref[...] = (row_ref[...] << 16) | col_ref[...]


def _pack(row, col):
    return pl.pallas_call(
        _pack_body,
        out_shape=jax.ShapeDtypeStruct((E,), jnp.int32),
    )(row, col)


def _deg_body(parts_ref, dis_ref, inv_ref):
    deg = jnp.sum(parts_ref[...], axis=0) + 1.0
    inv = 1.0 / deg
    inv_ref[...] = inv
    dis_ref[...] = jnp.sqrt(inv)


def _deg_finish(parts):
    return pl.pallas_call(
        _deg_body,
        out_shape=(
            jax.ShapeDtypeStruct((N,), jnp.float32),
            jax.ShapeDtypeStruct((N,), jnp.float32),
        ),
    )(parts)


def _pack_bf16_pairs(even, odd):
    a = lax.bitcast_convert_type(even.astype(jnp.bfloat16),
                                 jnp.uint16).astype(jnp.uint32)
    b = lax.bitcast_convert_type(odd.astype(jnp.bfloat16),
                                 jnp.uint16).astype(jnp.uint32)
    return lax.bitcast_convert_type((b << 16) | a, jnp.int32)


def _mm_body(x_ref, w_ref, we_ref, wo_ref, dis_ref, linT_ref, linsP_ref):
    dis = dis_ref[...].reshape(1, N)
    linT_ref[...] = lax.dot_general(
        w_ref[...], x_ref[...], (((0,), (1,)), ((), ())),
        preferred_element_type=jnp.float32)
    even = lax.dot_general(we_ref[...], x_ref[...], (((0,), (1,)), ((), ())),
                           preferred_element_type=jnp.float32) * dis
    odd = lax.dot_general(wo_ref[...], x_ref[...], (((0,), (1,)), ((), ())),
                          preferred_element_type=jnp.float32) * dis
    linsP_ref[...] = _pack_bf16_pairs(even, odd)


def _matmul(x, w, we, wo, dis):
    return pl.pallas_call(
        _mm_body,
        out_shape=(
            jax.ShapeDtypeStruct((D, N), jnp.float32),
            jax.ShapeDtypeStruct((D // 2, N), jnp.int32),
        ),
    )(x, w, we, wo, dis)


def _bn_relu_T(t, gamma, beta):
    m = jnp.mean(t, axis=1, keepdims=True)
    v = jnp.mean(t * t, axis=1, keepdims=True) - m * m
    h = (t - m) * lax.rsqrt(v + 1e-5) * gamma.reshape(D, 1) + beta.reshape(D, 1)
    return jnp.maximum(h, 0.0)


def _mid_body(accT_ref, linT_ref, dis_ref, inv_ref, b_ref, g_ref, be_ref,
              w2_ref, w2e_ref, w2o_ref, lin2T_ref, lin2sP_ref):
    dis = dis_ref[...].reshape(1, N)
    inv = inv_ref[...].reshape(1, N)
    t = dis * accT_ref[...] + inv * linT_ref[...] + b_ref[...].reshape(D, 1)
    h = _bn_relu_T(t, g_ref[...], be_ref[...])
    lin2T_ref[...] = lax.dot_general(w2_ref[...], h, (((0,), (0,)), ((), ())),
                                     preferred_element_type=jnp.float32)
    even = lax.dot_general(w2e_ref[...], h, (((0,), (0,)), ((), ())),
                           preferred_element_type=jnp.float32) * dis
    odd = lax.dot_general(w2o_ref[...], h, (((0,), (0,)), ((), ())),
                          preferred_element_type=jnp.float32) * dis
    lin2sP_ref[...] = _pack_bf16_pairs(even, odd)


def _layer_mid(accT, linT, dis, inv_deg, b, gamma, beta, w2, w2e, w2o):
    return pl.pallas_call(
        _mid_body,
        out_shape=(
            jax.ShapeDtypeStruct((D, N), jnp.float32),
            jax.ShapeDtypeStruct((D // 2, N), jnp.int32),
        ),
    )(accT, linT, dis, inv_deg, b, gamma, beta, w2, w2e, w2o)


def _out_body(accT_ref, linT_ref, dis_ref, inv_ref, b_ref, g_ref, be_ref, o_ref):
    dis = dis_ref[...].reshape(1, N)
    inv = inv_ref[...].reshape(1, N)
    t = dis * accT_ref[...] + inv * linT_ref[...] + b_ref[...].reshape(D, 1)
    h = _bn_relu_T(t, g_ref[...], be_ref[...])
    o_ref[...] = h.T


def _layer_out(accT, linT, dis, inv_deg, b, gamma, beta):
    return pl.pallas_call(
        _out_body,
        out_shape=jax.ShapeDtypeStruct((N, D), jnp.float32),
    )(accT, linT, dis, inv_deg, b, gamma, beta)


def kernel(x, edge_index, W1, b1, gamma1, beta1, W2, b2, gamma2, beta2):
    row = edge_index[0].astype(jnp.int32)
    col = edge_index[1].astype(jnp.int32)

    pk = _pack(row, col)
    parts = _hist_sc_kernel()(pk)
    dis, inv_deg = _deg_finish(parts)

    lin1T, lin1sP = _matmul(x, W1, W1[:, 0::2], W1[:, 1::2], dis)
    acc1T = _agg_sc_kernel()(pk, lin1sP.reshape(NW, N * PPT)).reshape(D, N)
    lin2T, lin2sP = _layer_mid(acc1T, lin1T, dis, inv_deg, b1, gamma1, beta1,
                               W2, W2[:, 0::2], W2[:, 1::2])
    acc2T = _agg_sc_kernel()(pk, lin2sP.reshape(NW, N * PPT)).reshape(D, N)
    return _layer_out(acc2T, lin2T, dis, inv_deg, b2, gamma2, beta2)


# speedup vs baseline: 1.1107x; 1.0049x over previous
import functools

import jax
import jax.numpy as jnp
from jax import lax
from jax.experimental import pallas as pl
from jax.experimental.pallas import tpu as pltpu
from jax.experimental.pallas import tpu_sc as plsc

N = 10000
E = 320000
D = 128

NC = 2
NS = 16
NW = NC * NS
FPT = D // NW
PPT = FPT // 2
EPW = E // NW
EC = 20000
L = 16
UNROLL = 4


@functools.cache
def _mesh():
    return plsc.VectorSubcoreMesh(core_axis_name="c", subcore_axis_name="s",
                                  num_cores=NC, num_subcores=NS)


def _wid():
    return lax.axis_index("s") * NC + lax.axis_index("c")


def _zero_fill(ref, nwords):
    z = jnp.zeros((L,), jnp.float32)

    def body(i, _):
        ref[pl.ds(i * L, L)] = z
        return 0

    lax.fori_loop(0, nwords // L, body, 0)


@functools.cache
def _hist_sc_kernel():
    return pl.kernel(
        _hist_sc_body,
        out_type=(
            jax.ShapeDtypeStruct((E,), jnp.int32),
            jax.ShapeDtypeStruct((NW, N), jnp.float32),
        ),
        mesh=_mesh(),
        scratch_types=[
            pltpu.VMEM((EPW,), jnp.int32),
            pltpu.VMEM((EPW,), jnp.int32),
            pltpu.VMEM((EPW,), jnp.int32),
            pltpu.VMEM((N,), jnp.float32),
        ],
        compiler_params=pltpu.CompilerParams(needs_layout_passes=False),
    )


def _hist_sc_body(row_hbm, col_hbm, pk_hbm, out_hbm, row_v, col_v, pk_v, hist_v):
    w = _wid()
    pltpu.sync_copy(row_hbm.at[pl.ds(w * EPW, EPW)], row_v)
    pltpu.sync_copy(col_hbm.at[pl.ds(w * EPW, EPW)], col_v)
    _zero_fill(hist_v, N)
    ones = jnp.ones((L,), jnp.float32)

    @plsc.parallel_loop(0, EPW // L, unroll=UNROLL)
    def body(j):
        sl = pl.ds(j * L, L)
        cols = col_v[sl]
        pk_v[sl] = (row_v[sl] << 16) | cols
        plsc.addupdate_scatter(hist_v, [cols], ones)

    pltpu.sync_copy(pk_v, pk_hbm.at[pl.ds(w * EPW, EPW)])
    pltpu.sync_copy(hist_v, out_hbm.at[w])


@functools.cache
def _agg_sc_kernel():
    return pl.kernel(
        _agg_sc_body,
        out_type=jax.ShapeDtypeStruct((NW, N * FPT), jnp.float32),
        mesh=_mesh(),
        scratch_types=[
            pltpu.VMEM((N * PPT,), jnp.int32),
            pltpu.VMEM((N * FPT,), jnp.float32),
            pltpu.VMEM((EC,), jnp.int32),
            pltpu.VMEM((EC,), jnp.int32),
            pltpu.SemaphoreType.DMA,
            pltpu.SemaphoreType.DMA,
        ],
        compiler_params=pltpu.CompilerParams(needs_layout_passes=False),
    )


def _agg_sc_body(pk_hbm, lin_hbm, out_hbm, lin_v, acc_v, pk_v0, pk_v1,
                 sem0, sem1):
    w = _wid()
    pltpu.sync_copy(lin_hbm.at[w], lin_v)
    _zero_fill(acc_v, N * FPT)
    lin_p = [lin_v.at[pl.ds(p * N, N)] for p in range(PPT)]
    acc_f = [acc_v.at[pl.ds(f * N, N)] for f in range(FPT)]
    hi_mask = jnp.full((L,), -65536, jnp.int32)
    bufs = [pk_v0, pk_v1]
    sems = [sem0, sem1]
    nchunk = E // EC

    cps = [pltpu.async_copy(pk_hbm.at[pl.ds(0, EC)], bufs[0], sems[0]), None]
    for c in range(nchunk):
        b = c & 1
        cps[b].wait()
        if c + 1 < nchunk:
            cps[1 - b] = pltpu.async_copy(
                pk_hbm.at[pl.ds((c + 1) * EC, EC)], bufs[1 - b], sems[1 - b])
        pk_v = bufs[b]

        @plsc.parallel_loop(0, EC // L, unroll=UNROLL)
        def grp(j):
            pk = pk_v[pl.ds(j * L, L)]
            rows = pk >> 16
            cols = pk & 0xFFFF
            for p in range(PPT):
                pair = plsc.load_gather(lin_p[p], [rows])
                lo = plsc.bitcast(pair << 16, jnp.float32)
                hi = plsc.bitcast(pair & hi_mask, jnp.float32)
                plsc.addupdate_scatter(acc_f[2 * p], [cols], lo)
                plsc.addupdate_scatter(acc_f[2 * p + 1], [cols], hi)

    pltpu.sync_copy(acc_v, out_hbm.at[w])


def _pack_bf16_pairs(even, odd):
    a = lax.bitcast_convert_type(even.astype(jnp.bfloat16),
                                 jnp.uint16).astype(jnp.uint32)
    b = lax.bitcast_convert_type(odd.astype(jnp.bfloat16),
                                 jnp.uint16).astype(jnp.uint32)
    return lax.bitcast_convert_type((b << 16) | a, jnp.int32)


def _mm_body(x_ref, w_ref, we_ref, wo_ref, parts_ref, linT_ref, linsP_ref,
             dis_ref, inv_ref):
    deg = jnp.sum(parts_ref[...], axis=0) + 1.0
    inv = 1.0 / deg
    dis = jnp.sqrt(inv)
    inv_ref[...] = inv
    dis_ref[...] = dis
    dis = dis.reshape(1, N)
    linT_ref[...] = lax.dot_general(
        w_ref[...], x_ref[...], (((0,), (1,)), ((), ())),
        preferred_element_type=jnp.float32)
    even = lax.dot_general(we_ref[...], x_ref[...], (((0,), (1,)), ((), ())),
                           preferred_element_type=jnp.float32) * dis
    odd = lax.dot_general(wo_ref[...], x_ref[...], (((0,), (1,)), ((), ())),
                          preferred_element_type=jnp.float32) * dis
    linsP_ref[...] = _pack_bf16_pairs(even, odd)


def _matmul(x, w, we, wo, parts):
    return pl.pallas_call(
        _mm_body,
        out_shape=(
            jax.ShapeDtypeStruct((D, N), jnp.float32),
            jax.ShapeDtypeStruct((D // 2, N), jnp.int32),
            jax.ShapeDtypeStruct((N,), jnp.float32),
            jax.ShapeDtypeStruct((N,), jnp.float32),
        ),
    )(x, w, we, wo, parts)


def _bn_relu_T(t, gamma, beta):
    m = jnp.mean(t, axis=1, keepdims=True)
    v = jnp.mean(t * t, axis=1, keepdims=True) - m * m
    h = (t - m) * lax.rsqrt(v + 1e-5) * gamma.reshape(D, 1) + beta.reshape(D, 1)
    return jnp.maximum(h, 0.0)


def _mid_body(accT_ref, linT_ref, dis_ref, inv_ref, b_ref, g_ref, be_ref,
              w2_ref, w2e_ref, w2o_ref, lin2T_ref, lin2sP_ref):
    dis = dis_ref[...].reshape(1, N)
    inv = inv_ref[...].reshape(1, N)
    t = dis * accT_ref[...] + inv * linT_ref[...] + b_ref[...].reshape(D, 1)
    h = _bn_relu_T(t, g_ref[...], be_ref[...])
    lin2T_ref[...] = lax.dot_general(w2_ref[...], h, (((0,), (0,)), ((), ())),
                                     preferred_element_type=jnp.float32)
    even = lax.dot_general(w2e_ref[...], h, (((0,), (0,)), ((), ())),
                           preferred_element_type=jnp.float32) * dis
    odd = lax.dot_general(w2o_ref[...], h, (((0,), (0,)), ((), ())),
                          preferred_element_type=jnp.float32) * dis
    lin2sP_ref[...] = _pack_bf16_pairs(even, odd)


def _layer_mid(accT, linT, dis, inv_deg, b, gamma, beta, w2, w2e, w2o):
    return pl.pallas_call(
        _mid_body,
        out_shape=(
            jax.ShapeDtypeStruct((D, N), jnp.float32),
            jax.ShapeDtypeStruct((D // 2, N), jnp.int32),
        ),
    )(accT, linT, dis, inv_deg, b, gamma, beta, w2, w2e, w2o)


def _out_body(accT_ref, linT_ref, dis_ref, inv_ref, b_ref, g_ref, be_ref, o_ref):
    dis = dis_ref[...].reshape(1, N)
    inv = inv_ref[...].reshape(1, N)
    t = dis * accT_ref[...] + inv * linT_ref[...] + b_ref[...].reshape(D, 1)
    h = _bn_relu_T(t, g_ref[...], be_ref[...])
    o_ref[...] = h.T


def _layer_out(accT, linT, dis, inv_deg, b, gamma, beta):
    return pl.pallas_call(
        _out_body,
        out_shape=jax.ShapeDtypeStruct((N, D), jnp.float32),
    )(accT, linT, dis, inv_deg, b, gamma, beta)


def kernel(x, edge_index, W1, b1, gamma1, beta1, W2, b2, gamma2, beta2):
    row = edge_index[0].astype(jnp.int32)
    col = edge_index[1].astype(jnp.int32)

    pk, parts = _hist_sc_kernel()(row, col)

    lin1T, lin1sP, dis, inv_deg = _matmul(x, W1, W1[:, 0::2], W1[:, 1::2], parts)
    acc1T = _agg_sc_kernel()(pk, lin1sP.reshape(NW, N * PPT)).reshape(D, N)
    lin2T, lin2sP = _layer_mid(acc1T, lin1T, dis, inv_deg, b1, gamma1, beta1,
                               W2, W2[:, 0::2], W2[:, 1::2])
    acc2T = _agg_sc_kernel()(pk, lin2sP.reshape(NW, N * PPT)).reshape(D, N)
    return _layer_out(acc2T, lin2T, dis, inv_deg, b2, gamma2, beta2)
